# Initial kernel scaffold; baseline (speedup 1.0000x reference)
import jax
import jax.numpy as jnp
from jax.experimental import pallas as pl


def kernel(x, edge_index, edge_weight, W1, b1, W2, b2, gamma1, beta1, gamma2, beta2):
    raise NotImplementedError("write your pallas kernel here")


# speedup vs baseline: 5.4482x; 5.4482x over previous
import functools

import jax
import jax.numpy as jnp
from jax import lax
from jax.experimental import pallas as pl
from jax.experimental.pallas import tpu as pltpu
from jax.experimental.pallas import tpu_sc as plsc

N = 10000
E = 320000
D = 128
EPS = 1e-5

NC = 2
NS = 16
NW = NC * NS

K = 80
EPW = E // NW
WPW = EPW // K
ZF = 640
ZL = N - (NS - 1) * ZF


def _sc_scatter_body(h_hbm, src1d, dst1d, ew1d, zeros_hbm, part,
                     src_l, ew_l, dst_win, rows_v, acc, sem):
    c = lax.axis_index("c")
    s = lax.axis_index("s")
    w = c * NS + s

    pltpu.sync_copy(src1d.at[pl.ds(w * EPW, EPW)], src_l)
    pltpu.sync_copy(ew1d.at[pl.ds(w * EPW, EPW)], ew_l)

    @pl.when(s < NS - 1)
    def _():
        pltpu.sync_copy(zeros_hbm, acc.at[pl.ds(s * ZF, ZF)])

    @pl.when(s == NS - 1)
    def _():
        pltpu.sync_copy(zeros_hbm.at[pl.ds(0, ZL)], acc.at[pl.ds(s * ZF, ZL)])

    plsc.subcore_barrier()

    zi = jnp.zeros((16,), jnp.int32)

    def window(i, carry):
        pltpu.async_copy(h_hbm.at[src_l.at[pl.ds(i * K, K)]], rows_v,
                         sem).wait()
        pltpu.sync_copy(dst1d.at[pl.ds(w * EPW + i * K, K)], dst_win)

        def group(g, carry2):
            chunk = ew_l[pl.ds(i * K + g * 16, 16)]
            for lane in range(16):
                ewb = lax.gather(
                    chunk, (zi + lane)[:, None],
                    lax.GatherDimensionNumbers(offset_dims=(),
                                               collapsed_slice_dims=(0,),
                                               start_index_map=(0,)),
                    slice_sizes=(1,),
                    mode=lax.GatherScatterMode.PROMISE_IN_BOUNDS)
                e = g * 16 + lane
                for f in range(D // 16):
                    sl = pl.ds(f * 16, 16)
                    rows_v[e, sl] = rows_v[e, sl] * ewb
            return carry2

        lax.fori_loop(0, K // 16, group, 0)

        pltpu.sync_copy(rows_v, acc.at[dst_win], add=True)
        return carry

    lax.fori_loop(0, WPW, window, 0)
    plsc.subcore_barrier()

    @pl.when(s < NS - 1)
    def _():
        pltpu.sync_copy(acc.at[pl.ds(s * ZF, ZF)],
                        part.at[c, pl.ds(s * ZF, ZF)])

    @pl.when(s == NS - 1)
    def _():
        pltpu.sync_copy(acc.at[pl.ds(s * ZF, ZL)],
                        part.at[c, pl.ds(s * ZF, ZL)])


_sc_scatter = functools.partial(
    pl.kernel,
    out_type=jax.ShapeDtypeStruct((NC, N, D), jnp.float32),
    mesh=plsc.VectorSubcoreMesh(core_axis_name="c", subcore_axis_name="s"),
    scratch_types=[
        pltpu.VMEM((EPW,), jnp.int32),
        pltpu.VMEM((EPW,), jnp.float32),
        pltpu.VMEM((K,), jnp.int32),
        pltpu.VMEM((K, D), jnp.float32),
        pltpu.VMEM_SHARED((N, D), jnp.float32),
        pltpu.SemaphoreType.DMA,
    ],
)(_sc_scatter_body)


def _mm_kernel(x_ref, w_ref, o_ref):
    o_ref[...] = jnp.dot(x_ref[...], w_ref[...],
                         preferred_element_type=jnp.float32)


def _act_mm_kernel(p_ref, b_ref, s_ref, t_ref, w_ref, o_ref):
    m = p_ref[0] + p_ref[1] + b_ref[...]
    a = jnp.maximum(m * s_ref[...] + t_ref[...], 0.0)
    o_ref[...] = jnp.dot(a, w_ref[...], preferred_element_type=jnp.float32)


def _act_kernel(p_ref, b_ref, s_ref, t_ref, o_ref):
    m = p_ref[0] + p_ref[1] + b_ref[...]
    o_ref[...] = jnp.maximum(m * s_ref[...] + t_ref[...], 0.0)


_MB = 1000
_GRID = (N // _MB,)


def _matmul(x, W):
    return pl.pallas_call(
        _mm_kernel,
        grid=_GRID,
        in_specs=[pl.BlockSpec((_MB, D), lambda i: (i, 0)),
                  pl.BlockSpec((D, D), lambda i: (0, 0))],
        out_specs=pl.BlockSpec((_MB, D), lambda i: (i, 0)),
        out_shape=jax.ShapeDtypeStruct((N, D), jnp.float32),
    )(x, W)


def _act_matmul(part, b, scale, beta, W):
    vec = pl.BlockSpec((1, D), lambda i: (0, 0))
    return pl.pallas_call(
        _act_mm_kernel,
        grid=_GRID,
        in_specs=[pl.BlockSpec((NC, _MB, D), lambda i: (0, i, 0)),
                  vec, vec, vec,
                  pl.BlockSpec((D, D), lambda i: (0, 0))],
        out_specs=pl.BlockSpec((_MB, D), lambda i: (i, 0)),
        out_shape=jax.ShapeDtypeStruct((N, D), jnp.float32),
    )(part, b, scale, beta, W)


def _act_only(part, b, scale, beta):
    vec = pl.BlockSpec((1, D), lambda i: (0, 0))
    return pl.pallas_call(
        _act_kernel,
        grid=_GRID,
        in_specs=[pl.BlockSpec((NC, _MB, D), lambda i: (0, i, 0)),
                  vec, vec, vec],
        out_specs=pl.BlockSpec((_MB, D), lambda i: (i, 0)),
        out_shape=jax.ShapeDtypeStruct((N, D), jnp.float32),
    )(part, b, scale, beta)


def kernel(x, edge_index, edge_weight, W1, b1, W2, b2,
           gamma1, beta1, gamma2, beta2):
    src = edge_index[0].astype(jnp.int32)
    dst = edge_index[1].astype(jnp.int32)
    eww = edge_weight.astype(jnp.float32)
    zeros = jnp.zeros((ZF, D), jnp.float32)

    inv = 1.0 / jnp.sqrt(jnp.float32(1.0) + EPS)
    s1 = (gamma1 * inv).reshape(1, D)
    s2 = (gamma2 * inv).reshape(1, D)
    b1r, t1 = b1.reshape(1, D), beta1.reshape(1, D)
    b2r, t2 = b2.reshape(1, D), beta2.reshape(1, D)

    h1 = _matmul(x, W1)
    p1 = _sc_scatter(h1, src, dst, eww, zeros)
    h2 = _act_matmul(p1, b1r, s1, t1, W2)
    p2 = _sc_scatter(h2, src, dst, eww, zeros)
    return _act_only(p2, b2r, s2, t2)


# speedup vs baseline: 9.3403x; 1.7144x over previous
import functools

import jax
import jax.numpy as jnp
from jax import lax
from jax.experimental import pallas as pl
from jax.experimental.pallas import tpu as pltpu
from jax.experimental.pallas import tpu_sc as plsc

N = 10000
E = 320000
D = 128
EPS = 1e-5

NC = 2
NS = 16
NW = NC * NS

K = 80
EPW = E // NW
WPW = EPW // K
ZF = 640
ZL = N - (NS - 1) * ZF


NBUF = 3
PFD = 2
TRIPS = WPW // NBUF
PEELED = WPW - TRIPS * NBUF


def _sc_scatter_body(h_hbm, src1d, dst1d, ew1d, zeros_hbm, part,
                     src_l,
                     d0, d1, d2,
                     e0, e1, e2,
                     r0, r1, r2,
                     acc,
                     gs0, gs1, gs2,
                     ss0, ss1, ss2,
                     ds0, ds1, ds2,
                     es0, es1, es2):
    dst_w = [d0, d1, d2]
    ew_w = [e0, e1, e2]
    rows = [r0, r1, r2]
    gsem = [gs0, gs1, gs2]
    ssem = [ss0, ss1, ss2]
    dsem = [ds0, ds1, ds2]
    esem = [es0, es1, es2]

    c = lax.axis_index("c")
    s = lax.axis_index("s")
    w = c * NS + s

    pltpu.sync_copy(src1d.at[pl.ds(w * EPW, EPW)], src_l)

    @pl.when(s < NS - 1)
    def _():
        pltpu.sync_copy(zeros_hbm, acc.at[pl.ds(s * ZF, ZF)])

    @pl.when(s == NS - 1)
    def _():
        pltpu.sync_copy(zeros_hbm.at[pl.ds(0, ZL)], acc.at[pl.ds(s * ZF, ZL)])

    plsc.subcore_barrier()

    zi = jnp.zeros((16,), jnp.int32)

    def fetch_start(i, b):
        pltpu.async_copy(dst1d.at[pl.ds(w * EPW + i * K, K)], dst_w[b],
                         dsem[b])
        pltpu.async_copy(ew1d.at[pl.ds(w * EPW + i * K, K)], ew_w[b],
                         esem[b])
        pltpu.async_copy(h_hbm.at[src_l.at[pl.ds(i * K, K)]], rows[b],
                         gsem[b])

    def scatter_wait(b):
        pltpu.make_async_copy(rows[b], acc.at[dst_w[b]], ssem[b]).wait()

    def process(b):
        pltpu.make_async_copy(h_hbm.at[src_l.at[pl.ds(0, K)]], rows[b],
                              gsem[b]).wait()
        pltpu.make_async_copy(dst1d.at[pl.ds(0, K)], dst_w[b],
                              dsem[b]).wait()
        pltpu.make_async_copy(ew1d.at[pl.ds(0, K)], ew_w[b],
                              esem[b]).wait()

        for g in range(K // 16):
            chunk = ew_w[b][pl.ds(g * 16, 16)]
            for lane in range(16):
                ewb = lax.gather(
                    chunk, (zi + lane)[:, None],
                    lax.GatherDimensionNumbers(offset_dims=(),
                                               collapsed_slice_dims=(0,),
                                               start_index_map=(0,)),
                    slice_sizes=(1,),
                    mode=lax.GatherScatterMode.PROMISE_IN_BOUNDS)
                e = g * 16 + lane
                for f in range(D // 16):
                    sl = pl.ds(f * 16, 16)
                    rows[b][e, sl] = rows[b][e, sl] * ewb

        pltpu.async_copy(rows[b], acc.at[dst_w[b]], ssem[b], add=True)

    for b in range(PFD):
        fetch_start(b, b)

    def trip(t, carry):
        for b in range(NBUF):
            i = t * NBUF + b
            process(b)

            j = i + PFD
            bj = (b + PFD) % NBUF

            @pl.when(j >= NBUF)
            def _(bj=bj):
                scatter_wait(bj)

            fetch_start(j, bj)
        return carry

    lax.fori_loop(0, TRIPS, trip, 0)

    for p in range(PEELED):
        process((TRIPS * NBUF + p) % NBUF)

    for b in range(NBUF):
        scatter_wait(b)
    plsc.subcore_barrier()

    @pl.when(s < NS - 1)
    def _():
        pltpu.sync_copy(acc.at[pl.ds(s * ZF, ZF)],
                        part.at[c, pl.ds(s * ZF, ZF)])

    @pl.when(s == NS - 1)
    def _():
        pltpu.sync_copy(acc.at[pl.ds(s * ZF, ZL)],
                        part.at[c, pl.ds(s * ZF, ZL)])


_sc_scatter = functools.partial(
    pl.kernel,
    out_type=jax.ShapeDtypeStruct((NC, N, D), jnp.float32),
    mesh=plsc.VectorSubcoreMesh(core_axis_name="c", subcore_axis_name="s"),
    scratch_types=(
        [pltpu.VMEM((EPW,), jnp.int32)]
        + [pltpu.VMEM((K,), jnp.int32) for _ in range(NBUF)]
        + [pltpu.VMEM((K,), jnp.float32) for _ in range(NBUF)]
        + [pltpu.VMEM((K, D), jnp.float32) for _ in range(NBUF)]
        + [pltpu.VMEM_SHARED((N, D), jnp.float32)]
        + [pltpu.SemaphoreType.DMA for _ in range(4 * NBUF)]
    ),
)(_sc_scatter_body)


def _mm_kernel(x_ref, w_ref, o_ref):
    o_ref[...] = jnp.dot(x_ref[...], w_ref[...],
                         preferred_element_type=jnp.float32)


def _act_mm_kernel(p_ref, b_ref, s_ref, t_ref, w_ref, o_ref):
    m = p_ref[0] + p_ref[1] + b_ref[...]
    a = jnp.maximum(m * s_ref[...] + t_ref[...], 0.0)
    o_ref[...] = jnp.dot(a, w_ref[...], preferred_element_type=jnp.float32)


def _act_kernel(p_ref, b_ref, s_ref, t_ref, o_ref):
    m = p_ref[0] + p_ref[1] + b_ref[...]
    o_ref[...] = jnp.maximum(m * s_ref[...] + t_ref[...], 0.0)


_MB = 1000
_GRID = (N // _MB,)


def _matmul(x, W):
    return pl.pallas_call(
        _mm_kernel,
        grid=_GRID,
        in_specs=[pl.BlockSpec((_MB, D), lambda i: (i, 0)),
                  pl.BlockSpec((D, D), lambda i: (0, 0))],
        out_specs=pl.BlockSpec((_MB, D), lambda i: (i, 0)),
        out_shape=jax.ShapeDtypeStruct((N, D), jnp.float32),
    )(x, W)


def _act_matmul(part, b, scale, beta, W):
    vec = pl.BlockSpec((1, D), lambda i: (0, 0))
    return pl.pallas_call(
        _act_mm_kernel,
        grid=_GRID,
        in_specs=[pl.BlockSpec((NC, _MB, D), lambda i: (0, i, 0)),
                  vec, vec, vec,
                  pl.BlockSpec((D, D), lambda i: (0, 0))],
        out_specs=pl.BlockSpec((_MB, D), lambda i: (i, 0)),
        out_shape=jax.ShapeDtypeStruct((N, D), jnp.float32),
    )(part, b, scale, beta, W)


def _act_only(part, b, scale, beta):
    vec = pl.BlockSpec((1, D), lambda i: (0, 0))
    return pl.pallas_call(
        _act_kernel,
        grid=_GRID,
        in_specs=[pl.BlockSpec((NC, _MB, D), lambda i: (0, i, 0)),
                  vec, vec, vec],
        out_specs=pl.BlockSpec((_MB, D), lambda i: (i, 0)),
        out_shape=jax.ShapeDtypeStruct((N, D), jnp.float32),
    )(part, b, scale, beta)


def kernel(x, edge_index, edge_weight, W1, b1, W2, b2,
           gamma1, beta1, gamma2, beta2):
    src = edge_index[0].astype(jnp.int32)
    dst = edge_index[1].astype(jnp.int32)
    eww = edge_weight.astype(jnp.float32)
    zeros = jnp.zeros((ZF, D), jnp.float32)

    inv = 1.0 / jnp.sqrt(jnp.float32(1.0) + EPS)
    s1 = (gamma1 * inv).reshape(1, D)
    s2 = (gamma2 * inv).reshape(1, D)
    b1r, t1 = b1.reshape(1, D), beta1.reshape(1, D)
    b2r, t2 = b2.reshape(1, D), beta2.reshape(1, D)

    h1 = _matmul(x, W1)
    p1 = _sc_scatter(h1, src, dst, eww, zeros)
    h2 = _act_matmul(p1, b1r, s1, t1, W2)
    p2 = _sc_scatter(h2, src, dst, eww, zeros)
    return _act_only(p2, b2r, s2, t2)
